# baseline (speedup 1.0000x reference)
import functools

import jax
import jax.numpy as jnp
from jax.experimental import pallas as pl
from jax.experimental.pallas import tpu as pltpu

_N_MODELS = 8
_N_SPECIES = 4
_N_SHIFTS = 32
_AEV_DIM = _N_SPECIES * _N_SHIFTS
_H1 = 256
_H2 = 192
_MAX_DIM = 200
_RC = 5.2
_ETA = 19.7
_BLK = 8
_SB = 4


def _aev_body(c_ref, oh_ref, aev_ref):
    a = c_ref.shape[2]
    for i in range(_SB):
        c = c_ref[i]
        ci = c[:, :, None]
        cj = c[:, None, :]
        d = ci - cj
        r2 = jnp.sum(d * d, axis=0)
        r = jnp.sqrt(r2 + 1e-12)
        ii = jax.lax.broadcasted_iota(jnp.int32, (a, a), 0)
        jj = jax.lax.broadcasted_iota(jnp.int32, (a, a), 1)
        valid = (ii != jj) & (r < _RC)
        fc = jnp.where(valid, 0.5 * jnp.cos(jnp.pi * r / _RC) + 0.5, 0.0)
        kk = jax.lax.broadcasted_iota(
            jnp.int32, (1, _N_SHIFTS, 1), 1).astype(jnp.float32)
        shifts = 0.8 + (5.1 - 0.8) / (_N_SHIFTS - 1) * kk
        t = r[:, None, :] - shifts
        g = 0.25 * jnp.exp(-_ETA * t * t) * fc[:, None, :]
        g2 = g.reshape(a * _N_SHIFTS, a)
        red = jax.lax.dot_general(
            g2, oh_ref[...],
            dimension_numbers=(((1,), (0,)), ((), ())),
            preferred_element_type=jnp.float32,
        )
        aev_ref[i] = red.reshape(a, _N_SHIFTS, _N_SPECIES).astype(jnp.bfloat16)


def _mlp_body(spc_ref, gidx_ref, *refs):
    del spc_ref, gidx_ref
    xs = refs[:_BLK]
    w1_ref, b1_ref, w2_ref, b2_ref, out_ref = refs[_BLK:]
    x = jnp.concatenate([x[...] for x in xs], axis=1)
    s = x.shape[0]
    xr = x.reshape(s * _BLK, _AEV_DIM).astype(jnp.float32)
    for m in range(_N_MODELS):
        h = jnp.dot(xr, w1_ref[m, 0], preferred_element_type=jnp.float32)
        h = h + b1_ref[0, m][None, :]
        h = jnp.where(h > 0, h, jnp.exp(h) - 1.0)
        h = jnp.dot(h, w2_ref[m, 0], preferred_element_type=jnp.float32)
        h = h + b2_ref[0, m][None, :]
        h = jnp.where(h > 0, h, jnp.exp(h) - 1.0)
        out_ref[:, :, m, :] = h.reshape(s, _BLK, _H2).astype(jnp.bfloat16)


def _gather_body(q_ref, *refs):
    del q_ref
    ys = refs[:_BLK]
    out_ref = refs[_BLK]
    v = jnp.concatenate([y[...] for y in ys], axis=1)
    out_ref[:, :, :, :_H2] = v.astype(jnp.float32)
    out_ref[:, :, :, _H2:] = jnp.zeros(
        (v.shape[0], _BLK, _N_MODELS, _MAX_DIM - _H2), jnp.float32
    )


def _x_index(jj, b, spc_ref, gidx_ref):
    del spc_ref
    return (0, gidx_ref[b * _BLK + jj], 0, 0)


def _w_index(b, spc_ref, gidx_ref):
    del gidx_ref
    return (0, spc_ref[b], 0, 0)


def _b_index(b, spc_ref, gidx_ref):
    del gidx_ref
    return (spc_ref[b], 0, 0)


def _y_index(jj, i, q_ref):
    return (0, q_ref[i * _BLK + jj], 0, 0)


def kernel(species, coordinates, W1, b1, W2, b2):
    s, a = species.shape
    a_pad = ((a + (_N_SPECIES - 1) * (_BLK - 1)) + _BLK - 1) // _BLK * _BLK

    asp = species[0].astype(jnp.int32)
    order = jnp.argsort(asp, stable=True).astype(jnp.int32)
    eq = asp[None, :] == jnp.arange(_N_SPECIES, dtype=jnp.int32)[:, None]
    counts = jnp.sum(eq, axis=1).astype(jnp.int32)
    co = jnp.concatenate(
        [jnp.zeros((1,), jnp.int32), jnp.cumsum(counts)[:-1].astype(jnp.int32)]
    )
    pc = (counts + _BLK - 1) // _BLK * _BLK
    po = jnp.concatenate(
        [jnp.zeros((1,), jnp.int32), jnp.cumsum(pc)[:-1].astype(jnp.int32)]
    )
    po_end = jnp.cumsum(pc).astype(jnp.int32)
    tt = jnp.arange(a_pad, dtype=jnp.int32)
    grp = jnp.minimum(
        jnp.sum((tt[:, None] >= po_end[None, :]), axis=1).astype(jnp.int32),
        _N_SPECIES - 1,
    )
    last_ne = jnp.max(
        jnp.where(counts > 0, jnp.arange(_N_SPECIES, dtype=jnp.int32), -1))
    slot_sp = jnp.where(tt >= po_end[_N_SPECIES - 1], last_ne, grp)
    pos = jnp.minimum(tt - po[slot_sp], counts[slot_sp] - 1)
    gidx = order[jnp.clip(co[slot_sp] + pos, 0, a - 1)]
    spc_blk = slot_sp[::_BLK]
    inv = jnp.zeros((a,), jnp.int32).at[order].set(jnp.arange(a, dtype=jnp.int32))
    q_orig = (po[asp] + (inv - co[asp])).astype(jnp.int32)

    ct = jnp.swapaxes(coordinates, 1, 2)
    oht = eq.T.astype(jnp.float32)
    aev_raw = pl.pallas_call(
        _aev_body,
        grid=(s // _SB,),
        in_specs=[
            pl.BlockSpec((_SB, 3, a), lambda i: (i, 0, 0)),
            pl.BlockSpec((a, _N_SPECIES), lambda i: (0, 0)),
        ],
        out_specs=pl.BlockSpec(
            (_SB, a, _N_SHIFTS, _N_SPECIES), lambda i: (i, 0, 0, 0)),
        out_shape=jax.ShapeDtypeStruct((s, a, _N_SHIFTS, _N_SPECIES), jnp.bfloat16),
    )(ct, oht)
    aev = aev_raw.reshape(s, a, _AEV_DIM)

    w1p = (
        W1.reshape(_N_MODELS, _N_SPECIES, _N_SPECIES, _N_SHIFTS, _H1)
        .transpose(0, 1, 3, 2, 4)
        .reshape(_N_MODELS, _N_SPECIES, _AEV_DIM, _H1)
    )

    nblk = a_pad // _BLK
    aev4 = aev.reshape(s, a, 1, _AEV_DIM)
    x_specs = [
        pl.BlockSpec((s, 1, 1, _AEV_DIM), functools.partial(_x_index, jj))
        for jj in range(_BLK)
    ]
    b1s = jnp.swapaxes(b1, 0, 1)
    b2s = jnp.swapaxes(b2, 0, 1)
    grid_spec = pltpu.PrefetchScalarGridSpec(
        num_scalar_prefetch=2,
        grid=(nblk,),
        in_specs=[
            *x_specs,
            pl.BlockSpec((_N_MODELS, 1, _AEV_DIM, _H1), _w_index),
            pl.BlockSpec((1, _N_MODELS, _H1), _b_index),
            pl.BlockSpec((_N_MODELS, 1, _H1, _H2), _w_index),
            pl.BlockSpec((1, _N_MODELS, _H2), _b_index),
        ],
        out_specs=pl.BlockSpec(
            (s, _BLK, _N_MODELS, _H2), lambda b, *_: (0, b, 0, 0)),
    )
    y = pl.pallas_call(
        _mlp_body,
        grid_spec=grid_spec,
        out_shape=jax.ShapeDtypeStruct((s, a_pad, _N_MODELS, _H2), jnp.bfloat16),
    )(spc_blk, gidx, *([aev4] * _BLK), w1p, b1s, W2, b2s)

    y_specs = [
        pl.BlockSpec((s, 1, _N_MODELS, _H2), functools.partial(_y_index, jj))
        for jj in range(_BLK)
    ]
    out = pl.pallas_call(
        _gather_body,
        grid_spec=pltpu.PrefetchScalarGridSpec(
            num_scalar_prefetch=1,
            grid=(a // _BLK,),
            in_specs=y_specs,
            out_specs=pl.BlockSpec(
                (s, _BLK, _N_MODELS, _MAX_DIM), lambda i, q: (0, i, 0, 0)),
        ),
        out_shape=jax.ShapeDtypeStruct((s, a, _N_MODELS, _MAX_DIM), jnp.float32),
    )(q_orig, *([y] * _BLK))

    return species, out


# speedup vs baseline: 1.0001x; 1.0001x over previous
import functools

import jax
import jax.numpy as jnp
from jax.experimental import pallas as pl
from jax.experimental.pallas import tpu as pltpu

_N_MODELS = 8
_N_SPECIES = 4
_N_SHIFTS = 32
_AEV_DIM = _N_SPECIES * _N_SHIFTS
_H1 = 256
_H2 = 192
_MAX_DIM = 200
_RC = 5.2
_ETA = 19.7
_BLK = 8
_SB = 4


def _aev_body(c_ref, oh_ref, aev_ref):
    a = c_ref.shape[2]
    for i in range(_SB):
        c = c_ref[i]
        ci = c[:, :, None]
        cj = c[:, None, :]
        d = ci - cj
        r2 = jnp.sum(d * d, axis=0)
        r = jnp.sqrt(r2 + 1e-12)
        ii = jax.lax.broadcasted_iota(jnp.int32, (a, a), 0)
        jj = jax.lax.broadcasted_iota(jnp.int32, (a, a), 1)
        valid = (ii != jj) & (r < _RC)
        fc = jnp.where(valid, 0.5 * jnp.cos(jnp.pi * r / _RC) + 0.5, 0.0)
        kk = jax.lax.broadcasted_iota(
            jnp.int32, (1, _N_SHIFTS, 1), 1).astype(jnp.float32)
        shifts = 0.8 + (5.1 - 0.8) / (_N_SHIFTS - 1) * kk
        t = r[:, None, :] - shifts
        g = 0.25 * jnp.exp(-_ETA * t * t) * fc[:, None, :]
        g2 = g.reshape(a * _N_SHIFTS, a)
        red = jax.lax.dot_general(
            g2, oh_ref[...],
            dimension_numbers=(((1,), (0,)), ((), ())),
            preferred_element_type=jnp.float32,
        )
        aev_ref[i] = red.reshape(a, _N_SHIFTS, _N_SPECIES).astype(jnp.bfloat16)


def _mlp_body(spc_ref, gidx_ref, *refs):
    del spc_ref, gidx_ref
    xs = refs[:_BLK]
    w1_ref, b1_ref, w2_ref, b2_ref, out_ref = refs[_BLK:]
    s = xs[0].shape[0]
    xr = jnp.concatenate(
        [x[...].reshape(s, _AEV_DIM) for x in xs], axis=0
    ).astype(jnp.float32)
    for m in range(_N_MODELS):
        h = jnp.dot(xr, w1_ref[m, 0], preferred_element_type=jnp.float32)
        h = h + b1_ref[0, m][None, :]
        h = jnp.where(h > 0, h, jnp.exp(h) - 1.0)
        h = jnp.dot(h, w2_ref[m, 0], preferred_element_type=jnp.float32)
        h = h + b2_ref[0, m][None, :]
        h = jnp.where(h > 0, h, jnp.exp(h) - 1.0)
        out_ref[:, :, m, :] = h.reshape(_BLK, s, _H2).astype(jnp.bfloat16)


def _gather_body(q_ref, *refs):
    del q_ref
    ys = refs[:_BLK]
    out_ref = refs[_BLK]
    s = ys[0].shape[1]
    out_ref[:, :, :, _H2:] = jnp.zeros(
        (s, _BLK, _N_MODELS, _MAX_DIM - _H2), jnp.float32
    )
    for jj in range(_BLK):
        out_ref[:, jj, :, :_H2] = ys[jj][0].astype(jnp.float32)


def _x_index(jj, b, spc_ref, gidx_ref):
    del spc_ref
    return (0, gidx_ref[b * _BLK + jj], 0, 0)


def _w_index(b, spc_ref, gidx_ref):
    del gidx_ref
    return (0, spc_ref[b], 0, 0)


def _b_index(b, spc_ref, gidx_ref):
    del gidx_ref
    return (spc_ref[b], 0, 0)


def _y_index(jj, i, q_ref):
    return (q_ref[i * _BLK + jj], 0, 0, 0)


def kernel(species, coordinates, W1, b1, W2, b2):
    s, a = species.shape
    a_pad = ((a + (_N_SPECIES - 1) * (_BLK - 1)) + _BLK - 1) // _BLK * _BLK

    asp = species[0].astype(jnp.int32)
    order = jnp.argsort(asp, stable=True).astype(jnp.int32)
    eq = asp[None, :] == jnp.arange(_N_SPECIES, dtype=jnp.int32)[:, None]
    counts = jnp.sum(eq, axis=1).astype(jnp.int32)
    co = jnp.concatenate(
        [jnp.zeros((1,), jnp.int32), jnp.cumsum(counts)[:-1].astype(jnp.int32)]
    )
    pc = (counts + _BLK - 1) // _BLK * _BLK
    po = jnp.concatenate(
        [jnp.zeros((1,), jnp.int32), jnp.cumsum(pc)[:-1].astype(jnp.int32)]
    )
    po_end = jnp.cumsum(pc).astype(jnp.int32)
    tt = jnp.arange(a_pad, dtype=jnp.int32)
    grp = jnp.minimum(
        jnp.sum((tt[:, None] >= po_end[None, :]), axis=1).astype(jnp.int32),
        _N_SPECIES - 1,
    )
    last_ne = jnp.max(
        jnp.where(counts > 0, jnp.arange(_N_SPECIES, dtype=jnp.int32), -1))
    slot_sp = jnp.where(tt >= po_end[_N_SPECIES - 1], last_ne, grp)
    pos = jnp.minimum(tt - po[slot_sp], counts[slot_sp] - 1)
    gidx = order[jnp.clip(co[slot_sp] + pos, 0, a - 1)]
    spc_blk = slot_sp[::_BLK]
    inv = jnp.zeros((a,), jnp.int32).at[order].set(jnp.arange(a, dtype=jnp.int32))
    q_orig = (po[asp] + (inv - co[asp])).astype(jnp.int32)

    ct = jnp.swapaxes(coordinates, 1, 2)
    oht = eq.T.astype(jnp.float32)
    aev_raw = pl.pallas_call(
        _aev_body,
        grid=(s // _SB,),
        in_specs=[
            pl.BlockSpec((_SB, 3, a), lambda i: (i, 0, 0)),
            pl.BlockSpec((a, _N_SPECIES), lambda i: (0, 0)),
        ],
        out_specs=pl.BlockSpec(
            (_SB, a, _N_SHIFTS, _N_SPECIES), lambda i: (i, 0, 0, 0)),
        out_shape=jax.ShapeDtypeStruct((s, a, _N_SHIFTS, _N_SPECIES), jnp.bfloat16),
    )(ct, oht)
    aev = aev_raw.reshape(s, a, _AEV_DIM)

    w1p = (
        W1.reshape(_N_MODELS, _N_SPECIES, _N_SPECIES, _N_SHIFTS, _H1)
        .transpose(0, 1, 3, 2, 4)
        .reshape(_N_MODELS, _N_SPECIES, _AEV_DIM, _H1)
    )

    nblk = a_pad // _BLK
    aev4 = aev.reshape(s, a, 1, _AEV_DIM)
    x_specs = [
        pl.BlockSpec((s, 1, 1, _AEV_DIM), functools.partial(_x_index, jj))
        for jj in range(_BLK)
    ]
    b1s = jnp.swapaxes(b1, 0, 1)
    b2s = jnp.swapaxes(b2, 0, 1)
    grid_spec = pltpu.PrefetchScalarGridSpec(
        num_scalar_prefetch=2,
        grid=(nblk,),
        in_specs=[
            *x_specs,
            pl.BlockSpec((_N_MODELS, 1, _AEV_DIM, _H1), _w_index),
            pl.BlockSpec((1, _N_MODELS, _H1), _b_index),
            pl.BlockSpec((_N_MODELS, 1, _H1, _H2), _w_index),
            pl.BlockSpec((1, _N_MODELS, _H2), _b_index),
        ],
        out_specs=pl.BlockSpec(
            (_BLK, s, _N_MODELS, _H2), lambda b, *_: (b, 0, 0, 0)),
    )
    y = pl.pallas_call(
        _mlp_body,
        grid_spec=grid_spec,
        out_shape=jax.ShapeDtypeStruct((a_pad, s, _N_MODELS, _H2), jnp.bfloat16),
    )(spc_blk, gidx, *([aev4] * _BLK), w1p, b1s, W2, b2s)

    y_specs = [
        pl.BlockSpec((1, s, _N_MODELS, _H2), functools.partial(_y_index, jj))
        for jj in range(_BLK)
    ]
    out = pl.pallas_call(
        _gather_body,
        grid_spec=pltpu.PrefetchScalarGridSpec(
            num_scalar_prefetch=1,
            grid=(a // _BLK,),
            in_specs=y_specs,
            out_specs=pl.BlockSpec(
                (s, _BLK, _N_MODELS, _MAX_DIM), lambda i, q: (0, i, 0, 0)),
        ),
        out_shape=jax.ShapeDtypeStruct((s, a, _N_MODELS, _MAX_DIM), jnp.float32),
    )(q_orig, *([y] * _BLK))

    return species, out


# speedup vs baseline: 1.0111x; 1.0109x over previous
import functools

import jax
import jax.numpy as jnp
from jax.experimental import pallas as pl
from jax.experimental.pallas import tpu as pltpu

_N_MODELS = 8
_N_SPECIES = 4
_N_SHIFTS = 32
_AEV_DIM = _N_SPECIES * _N_SHIFTS
_H1 = 256
_H2 = 192
_MAX_DIM = 200
_RC = 5.2
_ETA = 19.7
_BLK = 8
_SB = 4


def _aev_body(c_ref, oh_ref, aev_ref):
    a = c_ref.shape[2]
    for i in range(_SB):
        c = c_ref[i]
        ci = c[:, :, None]
        cj = c[:, None, :]
        d = ci - cj
        r2 = jnp.sum(d * d, axis=0)
        r = jnp.sqrt(r2 + 1e-12)
        ii = jax.lax.broadcasted_iota(jnp.int32, (a, a), 0)
        jj = jax.lax.broadcasted_iota(jnp.int32, (a, a), 1)
        valid = (ii != jj) & (r < _RC)
        fc = jnp.where(valid, 0.5 * jnp.cos(jnp.pi * r / _RC) + 0.5, 0.0)
        kk = jax.lax.broadcasted_iota(
            jnp.int32, (1, _N_SHIFTS, 1), 1).astype(jnp.float32)
        shifts = 0.8 + (5.1 - 0.8) / (_N_SHIFTS - 1) * kk
        t = r[:, None, :] - shifts
        g = 0.25 * jnp.exp(-_ETA * t * t) * fc[:, None, :]
        g2 = g.reshape(a * _N_SHIFTS, a)
        red = jax.lax.dot_general(
            g2, oh_ref[...],
            dimension_numbers=(((1,), (0,)), ((), ())),
            preferred_element_type=jnp.float32,
        )
        aev_ref[i] = red.reshape(a, _N_SHIFTS, _N_SPECIES)


def _mlp_body(spc_ref, gidx_ref, *refs):
    del spc_ref, gidx_ref
    xs = refs[:_BLK]
    w1_ref, b1_ref, w2_ref, b2_ref, out_ref = refs[_BLK:]
    x = jnp.concatenate([x[...] for x in xs], axis=1)
    s = x.shape[0]
    xr = x.reshape(s * _BLK, _AEV_DIM)
    out_ref[:, :, :, _H2:] = jnp.zeros(
        (s, _BLK, _N_MODELS, _MAX_DIM - _H2), jnp.float32
    )
    for m in range(_N_MODELS):
        h = jnp.dot(xr, w1_ref[m, 0], preferred_element_type=jnp.float32)
        h = h + b1_ref[0, m][None, :]
        h = jnp.where(h > 0, h, jnp.exp(h) - 1.0)
        h = jnp.dot(h, w2_ref[m, 0], preferred_element_type=jnp.float32)
        h = h + b2_ref[0, m][None, :]
        h = jnp.where(h > 0, h, jnp.exp(h) - 1.0)
        out_ref[:, :, m, :_H2] = h.reshape(s, _BLK, _H2)


def _gather_body(q_ref, *refs):
    del q_ref
    ys = refs[:_BLK]
    out_ref = refs[_BLK]
    out_ref[...] = jnp.concatenate([y[...] for y in ys], axis=1)


def _x_index(jj, b, spc_ref, gidx_ref):
    del spc_ref
    return (0, gidx_ref[b * _BLK + jj], 0, 0)


def _w_index(b, spc_ref, gidx_ref):
    del gidx_ref
    return (0, spc_ref[b], 0, 0)


def _b_index(b, spc_ref, gidx_ref):
    del gidx_ref
    return (spc_ref[b], 0, 0)


def _y_index(jj, i, q_ref):
    return (0, q_ref[i * _BLK + jj], 0, 0)


def kernel(species, coordinates, W1, b1, W2, b2):
    s, a = species.shape
    a_pad = ((a + (_N_SPECIES - 1) * (_BLK - 1)) + _BLK - 1) // _BLK * _BLK

    asp = species[0].astype(jnp.int32)
    order = jnp.argsort(asp, stable=True).astype(jnp.int32)
    eq = asp[None, :] == jnp.arange(_N_SPECIES, dtype=jnp.int32)[:, None]
    counts = jnp.sum(eq, axis=1).astype(jnp.int32)
    co = jnp.concatenate(
        [jnp.zeros((1,), jnp.int32), jnp.cumsum(counts)[:-1].astype(jnp.int32)]
    )
    pc = (counts + _BLK - 1) // _BLK * _BLK
    po = jnp.concatenate(
        [jnp.zeros((1,), jnp.int32), jnp.cumsum(pc)[:-1].astype(jnp.int32)]
    )
    po_end = jnp.cumsum(pc).astype(jnp.int32)
    tt = jnp.arange(a_pad, dtype=jnp.int32)
    grp = jnp.minimum(
        jnp.sum((tt[:, None] >= po_end[None, :]), axis=1).astype(jnp.int32),
        _N_SPECIES - 1,
    )
    last_ne = jnp.max(
        jnp.where(counts > 0, jnp.arange(_N_SPECIES, dtype=jnp.int32), -1))
    slot_sp = jnp.where(tt >= po_end[_N_SPECIES - 1], last_ne, grp)
    pos = jnp.minimum(tt - po[slot_sp], counts[slot_sp] - 1)
    gidx = order[jnp.clip(co[slot_sp] + pos, 0, a - 1)]
    spc_blk = slot_sp[::_BLK]
    inv = jnp.zeros((a,), jnp.int32).at[order].set(jnp.arange(a, dtype=jnp.int32))
    q_orig = (po[asp] + (inv - co[asp])).astype(jnp.int32)

    ct = jnp.swapaxes(coordinates, 1, 2)
    oht = eq.T.astype(jnp.float32)
    aev_raw = pl.pallas_call(
        _aev_body,
        grid=(s // _SB,),
        in_specs=[
            pl.BlockSpec((_SB, 3, a), lambda i: (i, 0, 0)),
            pl.BlockSpec((a, _N_SPECIES), lambda i: (0, 0)),
        ],
        out_specs=pl.BlockSpec(
            (_SB, a, _N_SHIFTS, _N_SPECIES), lambda i: (i, 0, 0, 0)),
        out_shape=jax.ShapeDtypeStruct((s, a, _N_SHIFTS, _N_SPECIES), jnp.float32),
    )(ct, oht)
    aev = aev_raw.reshape(s, a, _AEV_DIM)

    w1p = (
        W1.reshape(_N_MODELS, _N_SPECIES, _N_SPECIES, _N_SHIFTS, _H1)
        .transpose(0, 1, 3, 2, 4)
        .reshape(_N_MODELS, _N_SPECIES, _AEV_DIM, _H1)
    )

    nblk = a_pad // _BLK
    aev4 = aev.reshape(s, a, 1, _AEV_DIM)
    x_specs = [
        pl.BlockSpec((s, 1, 1, _AEV_DIM), functools.partial(_x_index, jj))
        for jj in range(_BLK)
    ]
    b1s = jnp.swapaxes(b1, 0, 1)
    b2s = jnp.swapaxes(b2, 0, 1)
    grid_spec = pltpu.PrefetchScalarGridSpec(
        num_scalar_prefetch=2,
        grid=(nblk,),
        in_specs=[
            *x_specs,
            pl.BlockSpec((_N_MODELS, 1, _AEV_DIM, _H1), _w_index),
            pl.BlockSpec((1, _N_MODELS, _H1), _b_index),
            pl.BlockSpec((_N_MODELS, 1, _H1, _H2), _w_index),
            pl.BlockSpec((1, _N_MODELS, _H2), _b_index),
        ],
        out_specs=pl.BlockSpec(
            (s, _BLK, _N_MODELS, _MAX_DIM), lambda b, *_: (0, b, 0, 0)),
    )
    y = pl.pallas_call(
        _mlp_body,
        grid_spec=grid_spec,
        out_shape=jax.ShapeDtypeStruct((s, a_pad, _N_MODELS, _MAX_DIM), jnp.float32),
    )(spc_blk, gidx, *([aev4] * _BLK), w1p, b1s, W2, b2s)

    y_specs = [
        pl.BlockSpec((s, 1, _N_MODELS, _MAX_DIM), functools.partial(_y_index, jj))
        for jj in range(_BLK)
    ]
    out = pl.pallas_call(
        _gather_body,
        grid_spec=pltpu.PrefetchScalarGridSpec(
            num_scalar_prefetch=1,
            grid=(a // _BLK,),
            in_specs=y_specs,
            out_specs=pl.BlockSpec(
                (s, _BLK, _N_MODELS, _MAX_DIM), lambda i, q: (0, i, 0, 0)),
        ),
        out_shape=jax.ShapeDtypeStruct((s, a, _N_MODELS, _MAX_DIM), jnp.float32),
    )(q_orig, *([y] * _BLK))

    return species, out
